# Initial kernel scaffold; baseline (speedup 1.0000x reference)
import jax
import jax.numpy as jnp
from jax.experimental import pallas as pl


def kernel(q, S, av_num):
    raise NotImplementedError("write your pallas kernel here")


# speedup vs baseline: 114.6864x; 114.6864x over previous
import functools

import jax
import jax.numpy as jnp
from jax.experimental import pallas as pl
from jax.experimental.pallas import tpu as pltpu

NEIGHBOR_K = 3.0


def _top3_rowsum(x):
    m1 = jnp.max(x, axis=1, keepdims=True)
    c1 = jnp.sum((x == m1).astype(jnp.float32), axis=1, keepdims=True)
    x2 = jnp.where(x < m1, x, -2.0)
    m2 = jnp.max(x2, axis=1, keepdims=True)
    c2 = jnp.sum((x2 == m2).astype(jnp.float32), axis=1, keepdims=True)
    x3 = jnp.where(x2 < m2, x2, -2.0)
    m3 = jnp.max(x3, axis=1, keepdims=True)
    t1 = jnp.minimum(c1, NEIGHBOR_K)
    t2 = jnp.minimum(c2, NEIGHBOR_K - t1)
    t3 = jnp.maximum(NEIGHBOR_K - t1 - t2, 0.0)
    return m1 * t1 + m2 * t2 + m3 * t3


def _knn_body(n_way, q_ref, s_ref, o_ref, sn_ref):
    @pl.when(pl.program_id(0) == 0)
    def _():
        s = s_ref[...]
        sn_ref[...] = s * jax.lax.rsqrt(jnp.sum(s * s, axis=1, keepdims=True))

    qb = q_ref[0]
    qn = qb * jax.lax.rsqrt(jnp.sum(qb * qb, axis=0, keepdims=True))
    totals = []
    for n in range(n_way):
        inner = jax.lax.dot_general(
            qn, sn_ref[n],
            dimension_numbers=(((0,), (0,)), ((), ())),
            preferred_element_type=jnp.float32,
        )
        totals.append(jnp.sum(_top3_rowsum(inner)))
    o_ref[...] = jnp.stack(totals)[None, None, :]


def kernel(q, S, av_num):
    B, C, H, W = q.shape
    HW = H * W
    n_way, _, M = S.shape
    q3 = q.reshape(B, C, HW)
    sim = pl.pallas_call(
        functools.partial(_knn_body, n_way),
        grid=(B,),
        in_specs=[
            pl.BlockSpec((1, C, HW), lambda i: (i, 0, 0)),
            pl.BlockSpec((n_way, C, M), lambda i: (0, 0, 0)),
        ],
        out_specs=pl.BlockSpec((1, 1, n_way), lambda i: (i, 0, 0)),
        out_shape=jax.ShapeDtypeStruct((B, 1, n_way), jnp.float32),
        scratch_shapes=[pltpu.VMEM((n_way, C, M), jnp.float32)],
    )(q3, S)
    sim = sim.reshape(B, n_way)
    g = sim.reshape(B, 1, n_way)
    pooled = jnp.exp(jnp.mean(jnp.log(g), axis=1))
    return jnp.where(jnp.asarray(av_num) > 1, pooled, sim)
